# baseline (speedup 1.0000x reference)
import functools

import jax
import jax.numpy as jnp
from jax import lax
from jax.experimental import pallas as pl
from jax.experimental.pallas import tpu as pltpu
from jax.experimental.pallas import tpu_sc as plsc

_VOCAB = 100000
_DIM = 768
_B = 4 * 2048
_NC = 1
_NS = 16
_NW = _NC * _NS
_BPW = _B // _NW
_CHUNK = 32
_NCHUNK = _BPW // _CHUNK
_NBUF = 4
_AHEAD = 2


@jax.jit
def _sc_gather3(W0, W1, W2, idx_flat):
    mesh = plsc.VectorSubcoreMesh(
        core_axis_name="c", subcore_axis_name="s", num_cores=_NC,
        num_subcores=_NS)
    out_type = [jax.ShapeDtypeStruct((_B, _DIM), jnp.float32)] * 3

    @functools.partial(
        pl.kernel,
        mesh=mesh,
        out_type=out_type,
        scratch_types=(
            [pltpu.VMEM((_BPW,), jnp.int32)]
            + [pltpu.VMEM((_CHUNK, _DIM), jnp.float32)] * _NBUF
            + [pltpu.SemaphoreType.DMA] * (2 * _NBUF)
        ),
    )
    def body(w0, w1, w2, idx_hbm, o0, o1, o2, idx_v, *rest):
        bufs = rest[:_NBUF]
        gsems = rest[_NBUF:2 * _NBUF]
        wsems = rest[2 * _NBUF:]
        wid = lax.axis_index("s") * _NC + lax.axis_index("c")
        base = wid * _BPW
        pltpu.sync_copy(idx_hbm.at[pl.ds(base, _BPW)], idx_v)

        tables = (w0, w1, w2)
        outs = (o0, o1, o2)
        tasks = [(t, c) for t in range(3) for c in range(_NCHUNK)]
        n = len(tasks)

        def start_gather(i):
            t, c = tasks[i]
            b = i % _NBUF
            return pltpu.async_copy(
                tables[t].at[idx_v.at[pl.ds(c * _CHUNK, _CHUNK)]],
                bufs[b], gsems[b])

        pend_g = [None] * _NBUF
        pend_w = [None] * _NBUF
        for j in range(min(_AHEAD, n)):
            pend_g[j % _NBUF] = start_gather(j)
        for i, (t, c) in enumerate(tasks):
            b = i % _NBUF
            pend_g[b].wait()
            pend_w[b] = pltpu.async_copy(
                bufs[b], outs[t].at[pl.ds(base + c * _CHUNK, _CHUNK)],
                wsems[b])
            k = i + _AHEAD
            if k < n:
                bk = k % _NBUF
                if pend_w[bk] is not None:
                    pend_w[bk].wait()
                pend_g[bk] = start_gather(k)
        for b in range(_NBUF):
            if pend_w[b] is not None:
                pend_w[b].wait()

    return body(W0, W1, W2, idx_flat)


def kernel(W0, W1, W2, inputs):
    idx_flat = inputs.reshape(-1).astype(jnp.int32)
    e0, e1, e2 = _sc_gather3(W0, W1, W2, idx_flat)
    shape = inputs.shape + (_DIM,)
    e0 = e0.reshape(shape)
    e1 = e1.reshape(shape)
    e2 = e2.reshape(shape)
    return (e0, e1, e2, e2, e1, e0)


# speedup vs baseline: 1.0604x; 1.0604x over previous
import functools

import jax
import jax.numpy as jnp
from jax import lax
from jax.experimental import pallas as pl
from jax.experimental.pallas import tpu as pltpu
from jax.experimental.pallas import tpu_sc as plsc

_VOCAB = 100000
_DIM = 768
_B = 4 * 2048
_NC = 2
_NS = 16
_NW = _NC * _NS
_BPW = _B // _NW
_CHUNK = 32
_NCHUNK = _BPW // _CHUNK
_NBUF = 4
_AHEAD = 2


def _make_gather():
    mesh = plsc.VectorSubcoreMesh(
        core_axis_name="c", subcore_axis_name="s", num_cores=_NC,
        num_subcores=_NS)

    @functools.partial(
        pl.kernel,
        mesh=mesh,
        out_type=jax.ShapeDtypeStruct((_B, _DIM), jnp.float32),
        scratch_types=(
            [pltpu.VMEM((_BPW,), jnp.int32)]
            + [pltpu.VMEM((_CHUNK, _DIM), jnp.float32)] * _NBUF
            + [pltpu.SemaphoreType.DMA] * (2 * _NBUF)
        ),
    )
    def body(table, idx_hbm, out, idx_v, *rest):
        bufs = rest[:_NBUF]
        gsems = rest[_NBUF:2 * _NBUF]
        wsems = rest[2 * _NBUF:]
        wid = lax.axis_index("s") * _NC + lax.axis_index("c")
        base = wid * _BPW
        pltpu.sync_copy(idx_hbm.at[pl.ds(base, _BPW)], idx_v)

        def start_gather(i):
            b = i % _NBUF
            return pltpu.async_copy(
                table.at[idx_v.at[pl.ds(i * _CHUNK, _CHUNK)]],
                bufs[b], gsems[b])

        pend_g = [None] * _NBUF
        pend_w = [None] * _NBUF
        for j in range(min(_AHEAD, _NCHUNK)):
            pend_g[j % _NBUF] = start_gather(j)
        for i in range(_NCHUNK):
            b = i % _NBUF
            pend_g[b].wait()
            pend_w[b] = pltpu.async_copy(
                bufs[b], out.at[pl.ds(base + i * _CHUNK, _CHUNK)],
                wsems[b])
            k = i + _AHEAD
            if k < _NCHUNK:
                bk = k % _NBUF
                if pend_w[bk] is not None:
                    pend_w[bk].wait()
                pend_g[bk] = start_gather(k)
        for b in range(_NBUF):
            if pend_w[b] is not None:
                pend_w[b].wait()

    return body


_gather = _make_gather()


@jax.jit
def _lookup_all(W0, W1, W2, idx_flat):
    e2 = _gather(W2, idx_flat)
    e1 = _gather(W1, idx_flat)
    e0 = _gather(W0, idx_flat)
    return e0, e1, e2


def kernel(W0, W1, W2, inputs):
    idx_flat = inputs.reshape(-1).astype(jnp.int32)
    e0, e1, e2 = _lookup_all(W0, W1, W2, idx_flat)
    shape = inputs.shape + (_DIM,)
    e0 = e0.reshape(shape)
    e1 = e1.reshape(shape)
    e2 = e2.reshape(shape)
    return (e0, e1, e2, e2, e1, e0)


# speedup vs baseline: 1.4110x; 1.3307x over previous
import functools

import jax
import jax.numpy as jnp
from jax import lax
from jax.experimental import pallas as pl
from jax.experimental.pallas import tpu as pltpu
from jax.experimental.pallas import tpu_sc as plsc

_VOCAB = 100000
_DIM = 768
_B = 4 * 2048
_NC = 2
_NS = 16
_NW = _NC * _NS
_BPW = _B // _NW
_CHUNK = 32
_NCHUNK = _BPW // _CHUNK
_NBUF = 4
_AHEAD = 2


@jax.jit
def _sc_gather3(W0, W1, W2, idx_flat):
    mesh = plsc.VectorSubcoreMesh(
        core_axis_name="c", subcore_axis_name="s", num_cores=_NC,
        num_subcores=_NS)
    out_type = [jax.ShapeDtypeStruct((_B, _DIM), jnp.float32)] * 6

    @functools.partial(
        pl.kernel,
        mesh=mesh,
        out_type=out_type,
        scratch_types=(
            [pltpu.VMEM((_BPW,), jnp.int32)]
            + [pltpu.VMEM((_CHUNK, _DIM), jnp.float32)] * _NBUF
            + [pltpu.SemaphoreType.DMA] * (3 * _NBUF)
        ),
    )
    def body(w0, w1, w2, idx_hbm, o0, o1, o2, o3, o4, o5, idx_v, *rest):
        bufs = rest[:_NBUF]
        gsems = rest[_NBUF:2 * _NBUF]
        wsems_a = rest[2 * _NBUF:3 * _NBUF]
        wsems_b = rest[3 * _NBUF:]
        wid = lax.axis_index("s") * _NC + lax.axis_index("c")
        base = wid * _BPW
        pltpu.sync_copy(idx_hbm.at[pl.ds(base, _BPW)], idx_v)

        tables = (w0, w1, w2)
        outs_a = (o0, o1, o2)
        outs_b = (o5, o4, o3)
        tasks = [(t, c) for t in range(3) for c in range(_NCHUNK)]
        n = len(tasks)

        def start_gather(i):
            t, c = tasks[i]
            b = i % _NBUF
            return pltpu.async_copy(
                tables[t].at[idx_v.at[pl.ds(c * _CHUNK, _CHUNK)]],
                bufs[b], gsems[b])

        pend_g = [None] * _NBUF
        pend_w = [None] * _NBUF
        for j in range(min(_AHEAD, n)):
            pend_g[j % _NBUF] = start_gather(j)
        for i, (t, c) in enumerate(tasks):
            b = i % _NBUF
            sl = pl.ds(base + c * _CHUNK, _CHUNK)
            pend_g[b].wait()
            wa = pltpu.async_copy(bufs[b], outs_a[t].at[sl], wsems_a[b])
            wb = pltpu.async_copy(bufs[b], outs_b[t].at[sl], wsems_b[b])
            pend_w[b] = (wa, wb)
            k = i + _AHEAD
            if k < n:
                bk = k % _NBUF
                if pend_w[bk] is not None:
                    pend_w[bk][0].wait()
                    pend_w[bk][1].wait()
                pend_g[bk] = start_gather(k)
        for b in range(_NBUF):
            if pend_w[b] is not None:
                pend_w[b][0].wait()
                pend_w[b][1].wait()

    return body(W0, W1, W2, idx_flat)


def kernel(W0, W1, W2, inputs):
    idx_flat = inputs.reshape(-1).astype(jnp.int32)
    outs = _sc_gather3(W0, W1, W2, idx_flat)
    shape = inputs.shape + (_DIM,)
    return tuple(o.reshape(shape) for o in outs)


# speedup vs baseline: 1.4201x; 1.0064x over previous
import functools

import jax
import jax.numpy as jnp
from jax import lax
from jax.experimental import pallas as pl
from jax.experimental.pallas import tpu as pltpu
from jax.experimental.pallas import tpu_sc as plsc

_VOCAB = 100000
_DIM = 768
_B = 4 * 2048
_NC = 2
_NS = 16
_NW = _NC * _NS
_BPW = _B // _NW
_CHUNK = 32
_NCHUNK = _BPW // _CHUNK
_NBUF = 5
_AHEAD = 3


@jax.jit
def _sc_gather3(W0, W1, W2, idx_flat):
    mesh = plsc.VectorSubcoreMesh(
        core_axis_name="c", subcore_axis_name="s", num_cores=_NC,
        num_subcores=_NS)
    out_type = [jax.ShapeDtypeStruct((_B, _DIM), jnp.float32)] * 6

    @functools.partial(
        pl.kernel,
        mesh=mesh,
        out_type=out_type,
        scratch_types=(
            [pltpu.VMEM((_BPW,), jnp.int32)]
            + [pltpu.VMEM((_CHUNK, _DIM), jnp.float32)] * _NBUF
            + [pltpu.SemaphoreType.DMA] * (3 * _NBUF)
        ),
    )
    def body(w0, w1, w2, idx_hbm, o0, o1, o2, o3, o4, o5, idx_v, *rest):
        bufs = rest[:_NBUF]
        gsems = rest[_NBUF:2 * _NBUF]
        wsems_a = rest[2 * _NBUF:3 * _NBUF]
        wsems_b = rest[3 * _NBUF:]
        wid = lax.axis_index("s") * _NC + lax.axis_index("c")
        base = wid * _BPW
        pltpu.sync_copy(idx_hbm.at[pl.ds(base, _BPW)], idx_v)

        tables = (w0, w1, w2)
        outs_a = (o0, o1, o2)
        outs_b = (o5, o4, o3)
        tasks = [(t, c) for t in range(3) for c in range(_NCHUNK)]
        n = len(tasks)

        def start_gather(i):
            t, c = tasks[i]
            b = i % _NBUF
            return pltpu.async_copy(
                tables[t].at[idx_v.at[pl.ds(c * _CHUNK, _CHUNK)]],
                bufs[b], gsems[b])

        pend_g = [None] * _NBUF
        pend_w = [None] * _NBUF
        for j in range(min(_AHEAD, n)):
            pend_g[j % _NBUF] = start_gather(j)
        for i, (t, c) in enumerate(tasks):
            b = i % _NBUF
            sl = pl.ds(base + c * _CHUNK, _CHUNK)
            pend_g[b].wait()
            wa = pltpu.async_copy(bufs[b], outs_a[t].at[sl], wsems_a[b])
            wb = pltpu.async_copy(bufs[b], outs_b[t].at[sl], wsems_b[b])
            pend_w[b] = (wa, wb)
            k = i + _AHEAD
            if k < n:
                bk = k % _NBUF
                if pend_w[bk] is not None:
                    pend_w[bk][0].wait()
                    pend_w[bk][1].wait()
                pend_g[bk] = start_gather(k)
        for b in range(_NBUF):
            if pend_w[b] is not None:
                pend_w[b][0].wait()
                pend_w[b][1].wait()

    return body(W0, W1, W2, idx_flat)


def kernel(W0, W1, W2, inputs):
    idx_flat = inputs.reshape(-1).astype(jnp.int32)
    outs = _sc_gather3(W0, W1, W2, idx_flat)
    shape = inputs.shape + (_DIM,)
    return tuple(o.reshape(shape) for o in outs)


# speedup vs baseline: 1.4230x; 1.0021x over previous
import functools

import jax
import jax.numpy as jnp
from jax import lax
from jax.experimental import pallas as pl
from jax.experimental.pallas import tpu as pltpu
from jax.experimental.pallas import tpu_sc as plsc

_VOCAB = 100000
_DIM = 768
_B = 4 * 2048
_NC = 2
_NS = 16
_NW = _NC * _NS
_BPW = _B // _NW
_CHUNK = 64
_NCHUNK = _BPW // _CHUNK
_NBUF = 2
_AHEAD = 1


@jax.jit
def _sc_gather3(W0, W1, W2, idx_flat):
    mesh = plsc.VectorSubcoreMesh(
        core_axis_name="c", subcore_axis_name="s", num_cores=_NC,
        num_subcores=_NS)
    out_type = [jax.ShapeDtypeStruct((_B, _DIM), jnp.float32)] * 6

    @functools.partial(
        pl.kernel,
        mesh=mesh,
        out_type=out_type,
        scratch_types=(
            [pltpu.VMEM((_BPW,), jnp.int32)]
            + [pltpu.VMEM((_CHUNK, _DIM), jnp.float32)] * _NBUF
            + [pltpu.SemaphoreType.DMA] * (3 * _NBUF)
        ),
    )
    def body(w0, w1, w2, idx_hbm, o0, o1, o2, o3, o4, o5, idx_v, *rest):
        bufs = rest[:_NBUF]
        gsems = rest[_NBUF:2 * _NBUF]
        wsems_a = rest[2 * _NBUF:3 * _NBUF]
        wsems_b = rest[3 * _NBUF:]
        wid = lax.axis_index("s") * _NC + lax.axis_index("c")
        base = wid * _BPW
        pltpu.sync_copy(idx_hbm.at[pl.ds(base, _BPW)], idx_v)

        tables = (w0, w1, w2)
        outs_a = (o0, o1, o2)
        outs_b = (o5, o4, o3)
        tasks = [(t, c) for t in range(3) for c in range(_NCHUNK)]
        n = len(tasks)

        def start_gather(i):
            t, c = tasks[i]
            b = i % _NBUF
            return pltpu.async_copy(
                tables[t].at[idx_v.at[pl.ds(c * _CHUNK, _CHUNK)]],
                bufs[b], gsems[b])

        pend_g = [None] * _NBUF
        pend_w = [None] * _NBUF
        for j in range(min(_AHEAD, n)):
            pend_g[j % _NBUF] = start_gather(j)
        for i, (t, c) in enumerate(tasks):
            b = i % _NBUF
            sl = pl.ds(base + c * _CHUNK, _CHUNK)
            pend_g[b].wait()
            wa = pltpu.async_copy(bufs[b], outs_a[t].at[sl], wsems_a[b])
            wb = pltpu.async_copy(bufs[b], outs_b[t].at[sl], wsems_b[b])
            pend_w[b] = (wa, wb)
            k = i + _AHEAD
            if k < n:
                bk = k % _NBUF
                if pend_w[bk] is not None:
                    pend_w[bk][0].wait()
                    pend_w[bk][1].wait()
                pend_g[bk] = start_gather(k)
        for b in range(_NBUF):
            if pend_w[b] is not None:
                pend_w[b][0].wait()
                pend_w[b][1].wait()

    return body(W0, W1, W2, idx_flat)


def kernel(W0, W1, W2, inputs):
    idx_flat = inputs.reshape(-1).astype(jnp.int32)
    outs = _sc_gather3(W0, W1, W2, idx_flat)
    shape = inputs.shape + (_DIM,)
    return tuple(o.reshape(shape) for o in outs)


# speedup vs baseline: 1.4235x; 1.0003x over previous
import functools

import jax
import jax.numpy as jnp
from jax import lax
from jax.experimental import pallas as pl
from jax.experimental.pallas import tpu as pltpu
from jax.experimental.pallas import tpu_sc as plsc

_VOCAB = 100000
_DIM = 768
_B = 4 * 2048
_NC = 2
_NS = 16
_NW = _NC * _NS
_BPW = _B // _NW
_CHUNK = 32
_NCHUNK = _BPW // _CHUNK
_NBUF = 3
_NSH = 2
_AHEAD = 2


@jax.jit
def _sc_gather3(W0, W1, W2, idx_flat):
    mesh = plsc.VectorSubcoreMesh(
        core_axis_name="c", subcore_axis_name="s", num_cores=_NC,
        num_subcores=_NS)
    out_type = [jax.ShapeDtypeStruct((_B, _DIM), jnp.float32)] * 6

    @functools.partial(
        pl.kernel,
        mesh=mesh,
        out_type=out_type,
        scratch_types=(
            [pltpu.VMEM((_BPW,), jnp.int32)]
            + [pltpu.VMEM((_CHUNK, _DIM), jnp.float32)] * _NBUF
            + [pltpu.VMEM_SHARED((_NS, _NSH, _CHUNK, _DIM), jnp.float32)]
            + [pltpu.SemaphoreType.DMA] * (2 * _NBUF + 2 * _NSH)
        ),
    )
    def body(w0, w1, w2, idx_hbm, o0, o1, o2, o3, o4, o5, idx_v, *rest):
        bufs = rest[:_NBUF]
        sh = rest[_NBUF]
        gsems = rest[_NBUF + 1:2 * _NBUF + 1]
        asems = rest[2 * _NBUF + 1:3 * _NBUF + 1]
        xsems = rest[3 * _NBUF + 1:3 * _NBUF + 1 + _NSH]
        ysems = rest[3 * _NBUF + 1 + _NSH:]
        sid = lax.axis_index("s")
        wid = sid * _NC + lax.axis_index("c")
        base = wid * _BPW
        pltpu.sync_copy(idx_hbm.at[pl.ds(base, _BPW)], idx_v)

        tables = (w0, w1, w2)
        outs_a = (o0, o1, o2)
        outs_b = (o5, o4, o3)
        tasks = [(t, c) for t in range(3) for c in range(_NCHUNK)]
        n = len(tasks)

        def start_gather(i):
            t, c = tasks[i]
            b = i % _NBUF
            return pltpu.async_copy(
                tables[t].at[idx_v.at[pl.ds(c * _CHUNK, _CHUNK)]],
                bufs[b], gsems[b])

        pend_g = [None] * _NBUF
        pend_a = [None] * _NBUF
        pend_x = [None] * _NSH
        pend_y = [None] * _NSH
        prev = None

        def issue_wy(s, t, c):
            pend_x[s].wait()
            pend_x[s] = None
            pend_y[s] = pltpu.async_copy(
                sh.at[sid, s],
                outs_b[t].at[pl.ds(base + c * _CHUNK, _CHUNK)], ysems[s])

        for j in range(min(_AHEAD, n)):
            pend_g[j % _NBUF] = start_gather(j)
        for i, (t, c) in enumerate(tasks):
            b = i % _NBUF
            if prev is not None:
                issue_wy(*prev)
                prev = None
            s = i % _NSH
            pend_g[b].wait()
            sl = pl.ds(base + c * _CHUNK, _CHUNK)
            pend_a[b] = pltpu.async_copy(bufs[b], outs_a[t].at[sl], asems[b])
            if pend_y[s] is not None:
                pend_y[s].wait()
                pend_y[s] = None
            pend_x[s] = pltpu.async_copy(bufs[b], sh.at[sid, s], xsems[s])
            prev = (s, t, c)
            k = i + _AHEAD
            if k < n:
                bk = k % _NBUF
                if pend_a[bk] is not None:
                    pend_a[bk].wait()
                    pend_a[bk] = None
                pend_g[bk] = start_gather(k)
        if prev is not None:
            issue_wy(*prev)
        for b in range(_NBUF):
            if pend_a[b] is not None:
                pend_a[b].wait()
        for s in range(_NSH):
            if pend_y[s] is not None:
                pend_y[s].wait()

    return body(W0, W1, W2, idx_flat)


def kernel(W0, W1, W2, inputs):
    idx_flat = inputs.reshape(-1).astype(jnp.int32)
    outs = _sc_gather3(W0, W1, W2, idx_flat)
    shape = inputs.shape + (_DIM,)
    return tuple(o.reshape(shape) for o in outs)
